# baseline (speedup 1.0000x reference)
import functools

import jax
import jax.numpy as jnp
from jax import lax
from jax.experimental import pallas as pl
from jax.experimental.pallas import tpu as pltpu
from jax.experimental.pallas import tpu_sc as plsc

VOCAB = 1000000
EMBED_DIM = 64
HIDDEN = 256
NUM_CLASSES = 50
BATCH = 4096
SEQ = 200

NC = 2
NS = 16
NW = NC * NS
BPW = BATCH // NW
CHUNK_A = 104
CHUNK_B = SEQ - CHUNK_A
INV_SEQ = 1.0 / SEQ


def _pool_body(x_hbm, el_hbm, er_hbm, out_hbm, idx_v, ra0, rb0, ra1, rb1,
               ra2, rb2, ra3, rb3, pooled_v,
               sa0, sb0, sa1, sb1, sa2, sb2, sa3, sb3):
    wid = lax.axis_index("s") * NC + lax.axis_index("c")
    base = wid * BPW
    pltpu.sync_copy(x_hbm.at[pl.ds(base * SEQ, BPW * SEQ)], idx_v)

    def dbl(k, _):
        idx_v[pl.ds(16 * k, 16)] = jnp.left_shift(idx_v[pl.ds(16 * k, 16)], 1)
        return 0

    lax.fori_loop(0, BPW * SEQ // 16, dbl, 0)

    def start_a(b, rows, sem):
        pltpu.async_copy(
            el_hbm.at[idx_v.at[pl.ds(b * SEQ, CHUNK_A)]], rows[0], sem)
        pltpu.async_copy(
            er_hbm.at[idx_v.at[pl.ds(b * SEQ, CHUNK_A)]], rows[1], sem)

    def start_b(b, rows, sem):
        pltpu.async_copy(
            el_hbm.at[idx_v.at[pl.ds(b * SEQ + CHUNK_A, CHUNK_B)]], rows[0],
            sem)
        pltpu.async_copy(
            er_hbm.at[idx_v.at[pl.ds(b * SEQ + CHUNK_A, CHUNK_B)]], rows[1],
            sem)

    start_a(0, ra0, sa0)
    start_b(0, rb0, sb0)
    start_a(1, ra1, sa1)
    start_b(1, rb1, sb1)
    start_a(2, ra2, sa2)
    start_b(2, rb2, sb2)
    start_a(3, ra3, sa3)
    start_b(3, rb3, sb3)

    def accum(rows, init, lo, hi):
        def j_body(j, acc):
            new = []
            for h in range(2):
                for i in range(2):
                    new.append(
                        acc[2 * h + i] + rows[h][j, pl.ds(16 * i, 16)])
            return tuple(new)
        return lax.fori_loop(lo, hi, j_body, init, unroll=8)

    def wait_a(rows, sem):
        for h in range(2):
            pltpu.make_async_copy(
                el_hbm.at[idx_v.at[pl.ds(0, CHUNK_A)]], rows[h], sem).wait()

    def wait_b(rows, sem):
        for h in range(2):
            pltpu.make_async_copy(
                el_hbm.at[idx_v.at[pl.ds(0, CHUNK_B)]], rows[h], sem).wait()

    zeros4 = tuple(jnp.zeros((16,), jnp.float32) for _ in range(4))

    def one_row(b, guard, ra, sa, rb, sb):
        wait_a(ra, sa)
        acc = accum(ra, zeros4, 0, CHUNK_A)

        @pl.when(guard)
        def _():
            start_a(b + 4, ra, sa)

        wait_b(rb, sb)
        acc = accum(rb, acc, 0, CHUNK_B)

        @pl.when(guard)
        def _():
            start_b(b + 4, rb, sb)

        for i in range(4):
            pooled_v[b, pl.ds(16 * i, 16)] = acc[i] * INV_SEQ

    def bb_body(bb, _):
        guard = bb < BPW // 4 - 1
        one_row(4 * bb, guard, ra0, sa0, rb0, sb0)
        one_row(4 * bb + 1, guard, ra1, sa1, rb1, sb1)
        one_row(4 * bb + 2, guard, ra2, sa2, rb2, sb2)
        one_row(4 * bb + 3, guard, ra3, sa3, rb3, sb3)
        return 0

    lax.fori_loop(0, BPW // 4, bb_body, 0)
    pltpu.sync_copy(pooled_v, out_hbm.at[pl.ds(base, BPW)])


@functools.partial(
    pl.kernel,
    out_type=jax.ShapeDtypeStruct((BATCH, EMBED_DIM), jnp.float32),
    mesh=plsc.VectorSubcoreMesh(core_axis_name="c", subcore_axis_name="s"),
    compiler_params=pltpu.CompilerParams(use_tc_tiling_on_sc=False),
    scratch_types=[
        pltpu.VMEM((BPW * SEQ,), jnp.int32),
        [pltpu.VMEM((CHUNK_A, 32), jnp.float32)] * 2,
        [pltpu.VMEM((CHUNK_B, 32), jnp.float32)] * 2,
        [pltpu.VMEM((CHUNK_A, 32), jnp.float32)] * 2,
        [pltpu.VMEM((CHUNK_B, 32), jnp.float32)] * 2,
        [pltpu.VMEM((CHUNK_A, 32), jnp.float32)] * 2,
        [pltpu.VMEM((CHUNK_B, 32), jnp.float32)] * 2,
        [pltpu.VMEM((CHUNK_A, 32), jnp.float32)] * 2,
        [pltpu.VMEM((CHUNK_B, 32), jnp.float32)] * 2,
        pltpu.VMEM((BPW, EMBED_DIM), jnp.float32),
    ] + [pltpu.SemaphoreType.DMA] * 8,
)
def _pool_sc(x_hbm, el_hbm, er_hbm, out_hbm, idx_v, ra0, rb0, ra1, rb1,
             ra2, rb2, ra3, rb3, pooled_v,
             sa0, sb0, sa1, sb1, sa2, sb2, sa3, sb3):
    _pool_body(x_hbm, el_hbm, er_hbm, out_hbm, idx_v, ra0, rb0, ra1, rb1,
               ra2, rb2, ra3, rb3, pooled_v,
               sa0, sb0, sa1, sb1, sa2, sb2, sa3, sb3)


def _mlp_body(p_ref, w1_ref, b1_ref, w2_ref, b2_ref, o_ref):
    h = jnp.dot(p_ref[...], w1_ref[...], preferred_element_type=jnp.float32)
    h = jnp.maximum(h + b1_ref[...], 0.0)
    o_ref[...] = (
        jnp.dot(h, w2_ref[...], preferred_element_type=jnp.float32)
        + b2_ref[...])


def _mlp_tc(pooled, W1, b1, W2p, b2p):
    return pl.pallas_call(
        _mlp_body,
        out_shape=jax.ShapeDtypeStruct((BATCH, 128), jnp.float32),
    )(pooled, W1, b1, W2p, b2p)


@jax.jit
def kernel(x, emb, W1, b1, W2, b2):
    x = x.astype(jnp.int32).reshape(BATCH * SEQ)
    el = jnp.pad(emb[:, :32], ((0, 0), (0, 32))).reshape(2 * VOCAB, 32)
    er = jnp.pad(emb[:, 32:], ((0, 0), (0, 32))).reshape(2 * VOCAB, 32)
    pooled = _pool_sc(x, el, er)

    W2p = jnp.pad(W2, ((0, 0), (0, 128 - NUM_CLASSES)))
    b2p = jnp.pad(b2, (0, 128 - NUM_CLASSES)).reshape(1, 128)
    out = _mlp_tc(pooled, W1, b1.reshape(1, HIDDEN), W2p, b2p)
    return out[:, :NUM_CLASSES]


# speedup vs baseline: 2.6669x; 2.6669x over previous
import functools

import jax
import jax.numpy as jnp
from jax import lax
from jax.experimental import pallas as pl
from jax.experimental.pallas import tpu as pltpu
from jax.experimental.pallas import tpu_sc as plsc

VOCAB = 1000000
EMBED_DIM = 64
HIDDEN = 256
NUM_CLASSES = 50
BATCH = 4096
SEQ = 200

NC = 2
NS = 16
NW = NC * NS
BPW = BATCH // NW
CHUNK_A = 104
CHUNK_B = SEQ - CHUNK_A
INV_SEQ = 1.0 / SEQ


def _pool_body(x_hbm, emb_hbm, out_hbm, idx_v, ra0, rb0, ra1, rb1,
               ra2, rb2, ra3, rb3, pooled_v,
               sa0, sb0, sa1, sb1, sa2, sb2, sa3, sb3):
    wid = lax.axis_index("s") * NC + lax.axis_index("c")
    base = wid * BPW
    pltpu.sync_copy(x_hbm.at[pl.ds(base * SEQ, BPW * SEQ)], idx_v)

    def dbl(k, _):
        idx_v[pl.ds(16 * k, 16)] = jnp.left_shift(idx_v[pl.ds(16 * k, 16)], 1)
        return 0

    lax.fori_loop(0, BPW * SEQ // 16, dbl, 0)

    def start_a(b, rows, sem):
        pltpu.async_copy(
            emb_hbm.at[idx_v.at[pl.ds(b * SEQ, CHUNK_A)]], rows, sem)

    def start_b(b, rows, sem):
        pltpu.async_copy(
            emb_hbm.at[idx_v.at[pl.ds(b * SEQ + CHUNK_A, CHUNK_B)]], rows,
            sem)

    start_a(0, ra0, sa0)
    start_b(0, rb0, sb0)
    start_a(1, ra1, sa1)
    start_b(1, rb1, sb1)
    start_a(2, ra2, sa2)
    start_b(2, rb2, sb2)
    start_a(3, ra3, sa3)
    start_b(3, rb3, sb3)

    def accum(rows, init, lo, hi):
        def j_body(j, acc):
            return tuple(
                acc[i] + rows[j, pl.ds(16 * i, 16)] for i in range(4))
        return lax.fori_loop(lo, hi, j_body, init, unroll=8)

    def wait_a(rows, sem):
        pltpu.make_async_copy(
            emb_hbm.at[idx_v.at[pl.ds(0, CHUNK_A)]], rows, sem).wait()

    def wait_b(rows, sem):
        pltpu.make_async_copy(
            emb_hbm.at[idx_v.at[pl.ds(0, CHUNK_B)]], rows, sem).wait()

    def one_row(b, guard, ra, sa, rb, sb):
        wait_a(ra, sa)
        acc = tuple(ra[0, pl.ds(16 * i, 16)] for i in range(4))
        acc = accum(ra, acc, 1, CHUNK_A)

        @pl.when(guard)
        def _():
            start_a(b + 4, ra, sa)

        wait_b(rb, sb)
        acc = accum(rb, acc, 0, CHUNK_B)

        @pl.when(guard)
        def _():
            start_b(b + 4, rb, sb)

        for i in range(4):
            pooled_v[b, pl.ds(16 * i, 16)] = acc[i] * INV_SEQ

    def bb_body(bb, _):
        guard = bb < BPW // 4 - 1
        one_row(4 * bb, guard, ra0, sa0, rb0, sb0)
        one_row(4 * bb + 1, guard, ra1, sa1, rb1, sb1)
        one_row(4 * bb + 2, guard, ra2, sa2, rb2, sb2)
        one_row(4 * bb + 3, guard, ra3, sa3, rb3, sb3)
        return 0

    lax.fori_loop(0, BPW // 4, bb_body, 0)
    pltpu.sync_copy(pooled_v, out_hbm.at[pl.ds(base, BPW)])


@functools.partial(
    pl.kernel,
    out_type=jax.ShapeDtypeStruct((BATCH, EMBED_DIM), jnp.float32),
    mesh=plsc.VectorSubcoreMesh(core_axis_name="c", subcore_axis_name="s"),
    compiler_params=pltpu.CompilerParams(use_tc_tiling_on_sc=False),
    scratch_types=[
        pltpu.VMEM((BPW * SEQ,), jnp.int32),
        pltpu.VMEM((CHUNK_A, EMBED_DIM), jnp.float32),
        pltpu.VMEM((CHUNK_B, EMBED_DIM), jnp.float32),
        pltpu.VMEM((CHUNK_A, EMBED_DIM), jnp.float32),
        pltpu.VMEM((CHUNK_B, EMBED_DIM), jnp.float32),
        pltpu.VMEM((CHUNK_A, EMBED_DIM), jnp.float32),
        pltpu.VMEM((CHUNK_B, EMBED_DIM), jnp.float32),
        pltpu.VMEM((CHUNK_A, EMBED_DIM), jnp.float32),
        pltpu.VMEM((CHUNK_B, EMBED_DIM), jnp.float32),
        pltpu.VMEM((BPW, EMBED_DIM), jnp.float32),
    ] + [pltpu.SemaphoreType.DMA] * 8,
)
def _pool_sc(x_hbm, emb_hbm, out_hbm, idx_v, ra0, rb0, ra1, rb1,
             ra2, rb2, ra3, rb3, pooled_v,
             sa0, sb0, sa1, sb1, sa2, sb2, sa3, sb3):
    _pool_body(x_hbm, emb_hbm, out_hbm, idx_v, ra0, rb0, ra1, rb1,
               ra2, rb2, ra3, rb3, pooled_v,
               sa0, sb0, sa1, sb1, sa2, sb2, sa3, sb3)


def _mlp_body(p_ref, w1_ref, b1_ref, w2_ref, b2_ref, o_ref):
    h = jnp.dot(p_ref[...], w1_ref[...], preferred_element_type=jnp.float32)
    h = jnp.maximum(h + b1_ref[...], 0.0)
    o_ref[...] = (
        jnp.dot(h, w2_ref[...], preferred_element_type=jnp.float32)
        + b2_ref[...])


def _mlp_tc(pooled, W1, b1, W2p, b2p):
    return pl.pallas_call(
        _mlp_body,
        out_shape=jax.ShapeDtypeStruct((BATCH, 128), jnp.float32),
    )(pooled, W1, b1, W2p, b2p)


@jax.jit
def kernel(x, emb, W1, b1, W2, b2):
    x = x.astype(jnp.int32).reshape(BATCH * SEQ)
    embp = jnp.pad(emb, ((0, 0), (0, 64))).reshape(2 * VOCAB, EMBED_DIM)
    pooled = _pool_sc(x, embp)

    W2p = jnp.pad(W2, ((0, 0), (0, 128 - NUM_CLASSES)))
    b2p = jnp.pad(b2, (0, 128 - NUM_CLASSES)).reshape(1, 128)
    out = _mlp_tc(pooled, W1, b1.reshape(1, HIDDEN), W2p, b2p)
    return out[:, :NUM_CLASSES]
